# baseline (speedup 1.0000x reference)
import functools

import jax
import jax.numpy as jnp
from jax import lax
from jax.experimental import pallas as pl
from jax.experimental.pallas import tpu as pltpu
from jax.experimental.pallas import tpu_sc as plsc

_NUM_TABLES = 4
_VOCAB = 1000000
_DIM = 32
_PER_KEY = 16384
_TOTAL = _NUM_TABLES * _PER_KEY
_QROWS = _VOCAB // 4
_QDIM = 4 * _DIM

_info = plsc.get_sparse_core_info()
_NC, _NS = _info.num_cores, _info.num_subcores
_NW = _NC * _NS
_B_PER_W = _PER_KEY // _NW
_CH = _B_PER_W // 2


def _grouped_gather(values, w0, w1, w2, w3):
    mesh = plsc.VectorSubcoreMesh(core_axis_name="c", subcore_axis_name="s")

    @functools.partial(
        pl.kernel,
        out_type=jax.ShapeDtypeStruct((_TOTAL, _DIM), jnp.float32),
        mesh=mesh,
        scratch_types=[
            pltpu.VMEM((_CH,), jnp.int32),
            pltpu.VMEM((_CH,), jnp.int32),
            pltpu.VMEM((_CH, _QDIM), jnp.float32),
            pltpu.VMEM((_CH, _DIM), jnp.float32),
            pltpu.SemaphoreType.DMA,
        ],
    )
    def k(values_hbm, w0_hbm, w1_hbm, w2_hbm, w3_hbm, out_hbm, idx_v, q_v,
          rows_v, out_v, sem):
        w = lax.axis_index("s") * _NC + lax.axis_index("c")
        for t, w_hbm in enumerate((w0_hbm, w1_hbm, w2_hbm, w3_hbm)):
            for h in range(_B_PER_W // _CH):
                base = t * _PER_KEY + w * _B_PER_W + h * _CH
                pltpu.sync_copy(values_hbm.at[pl.ds(base, _CH)], idx_v)

                def qbody(b, _):
                    r = idx_v[pl.ds(b * 16, 16)]
                    q_v[pl.ds(b * 16, 16)] = r >> 2
                    return 0

                lax.fori_loop(0, _CH // 16, qbody, 0)
                pltpu.async_copy(w_hbm.at[q_v], rows_v, sem).wait()

                def sbody(b, _):
                    offv = (idx_v[pl.ds(b * 16, 16)] & 3) * _DIM
                    for kk in range(16):
                        row = b * 16 + kk
                        off = offv[kk]
                        out_v[row, pl.ds(0, 16)] = rows_v[row, pl.ds(off, 16)]
                        out_v[row, pl.ds(16, 16)] = rows_v[row, pl.ds(off + 16, 16)]
                    return 0

                lax.fori_loop(0, _CH // 16, sbody, 0)
                pltpu.sync_copy(out_v, out_hbm.at[pl.ds(base, _CH)])

    return k(values, w0, w1, w2, w3)


@jax.jit
def kernel(values, W0, W1, W2, W3):
    return _grouped_gather(
        values,
        W0.reshape(_QROWS, _QDIM),
        W1.reshape(_QROWS, _QDIM),
        W2.reshape(_QROWS, _QDIM),
        W3.reshape(_QROWS, _QDIM),
    )


# speedup vs baseline: 1.5983x; 1.5983x over previous
import functools

import jax
import jax.numpy as jnp
from jax import lax
from jax.experimental import pallas as pl
from jax.experimental.pallas import tpu as pltpu
from jax.experimental.pallas import tpu_sc as plsc

_NUM_TABLES = 4
_VOCAB = 1000000
_DIM = 32
_PER_KEY = 16384
_TOTAL = _NUM_TABLES * _PER_KEY

_C = 8192
_C4 = _C // 4
_NB = (_VOCAB + _C - 1) // _C
_QROWS = _NB * _C4
_QDIM = 4 * _DIM

_info = plsc.get_sparse_core_info()
_NC, _NS = _info.num_cores, _info.num_subcores
_NW = _NC * _NS
_B_PER_W = _PER_KEY // _NW
_CH = _B_PER_W // 2


def _relayout(wt):
    def body(x_ref, o_ref):
        eye = jnp.eye(_DIM, dtype=jnp.float32)
        for m in range(4):
            o_ref[:, _DIM * m:_DIM * (m + 1)] = jax.lax.dot_general(
                x_ref[:, m * _C4:(m + 1) * _C4],
                eye,
                (((0,), (0,)), ((), ())),
                preferred_element_type=jnp.float32,
            )

    return pl.pallas_call(
        body,
        grid=(_NB,),
        in_specs=[pl.BlockSpec((_DIM, _C), lambda g: (0, g))],
        out_specs=pl.BlockSpec((_C4, _QDIM), lambda g: (g, 0)),
        out_shape=jax.ShapeDtypeStruct((_QROWS, _QDIM), jnp.float32),
    )(wt)


def _grouped_gather(values, w0, w1, w2, w3):
    mesh = plsc.VectorSubcoreMesh(core_axis_name="c", subcore_axis_name="s")

    @functools.partial(
        pl.kernel,
        out_type=jax.ShapeDtypeStruct((_TOTAL, _DIM), jnp.float32),
        mesh=mesh,
        scratch_types=[
            pltpu.VMEM((_CH,), jnp.int32),
            pltpu.VMEM((_CH,), jnp.int32),
            pltpu.VMEM((_CH, _QDIM), jnp.float32),
            pltpu.VMEM((_CH, _DIM), jnp.float32),
            pltpu.SemaphoreType.DMA,
        ],
    )
    def k(values_hbm, w0_hbm, w1_hbm, w2_hbm, w3_hbm, out_hbm, idx_v, q_v,
          rows_v, out_v, sem):
        w = lax.axis_index("s") * _NC + lax.axis_index("c")
        for t, w_hbm in enumerate((w0_hbm, w1_hbm, w2_hbm, w3_hbm)):
            for h in range(_B_PER_W // _CH):
                base = t * _PER_KEY + w * _B_PER_W + h * _CH
                pltpu.sync_copy(values_hbm.at[pl.ds(base, _CH)], idx_v)

                def qbody(b, _):
                    r = idx_v[pl.ds(b * 16, 16)]
                    q_v[pl.ds(b * 16, 16)] = ((r >> 13) << 11) + (r & 2047)
                    return 0

                lax.fori_loop(0, _CH // 16, qbody, 0)
                pltpu.async_copy(w_hbm.at[q_v], rows_v, sem).wait()

                def sbody(b, _):
                    offv = ((idx_v[pl.ds(b * 16, 16)] >> 11) & 3) * _DIM
                    for kk in range(16):
                        row = b * 16 + kk
                        off = offv[kk]
                        out_v[row, pl.ds(0, 16)] = rows_v[row, pl.ds(off, 16)]
                        out_v[row, pl.ds(16, 16)] = (
                            rows_v[row, pl.ds(off + 16, 16)]
                        )
                    return 0

                lax.fori_loop(0, _CH // 16, sbody, 0)
                pltpu.sync_copy(out_v, out_hbm.at[pl.ds(base, _CH)])

    return k(values, w0, w1, w2, w3)


@jax.jit
def kernel(values, W0, W1, W2, W3):
    return _grouped_gather(
        values,
        _relayout(W0.T),
        _relayout(W1.T),
        _relayout(W2.T),
        _relayout(W3.T),
    )


# speedup vs baseline: 1.6304x; 1.0200x over previous
import functools

import jax
import jax.numpy as jnp
from jax import lax
from jax.experimental import pallas as pl
from jax.experimental.pallas import tpu as pltpu
from jax.experimental.pallas import tpu_sc as plsc

_NUM_TABLES = 4
_VOCAB = 1000000
_DIM = 32
_PER_KEY = 16384
_TOTAL = _NUM_TABLES * _PER_KEY

_C = 32768
_C4 = _C // 4
_NB = (_VOCAB + _C - 1) // _C
_QROWS = _NB * _C4
_QDIM = 4 * _DIM

_info = plsc.get_sparse_core_info()
_NC, _NS = _info.num_cores, _info.num_subcores
_NW = _NC * _NS
_B_PER_W = _PER_KEY // _NW
_CH = _B_PER_W // 2


def _relayout(wt):
    def body(x_ref, o_ref):
        eye = jnp.eye(_DIM, dtype=jnp.float32)
        for m in range(4):
            o_ref[:, _DIM * m:_DIM * (m + 1)] = jax.lax.dot_general(
                x_ref[:, m * _C4:(m + 1) * _C4],
                eye,
                (((0,), (0,)), ((), ())),
                preferred_element_type=jnp.float32,
            )

    return pl.pallas_call(
        body,
        grid=(_NB,),
        in_specs=[pl.BlockSpec((_DIM, _C), lambda g: (0, g))],
        out_specs=pl.BlockSpec((_C4, _QDIM), lambda g: (g, 0)),
        out_shape=jax.ShapeDtypeStruct((_QROWS, _QDIM), jnp.float32),
    )(wt)


def _grouped_gather(values, w0, w1, w2, w3):
    mesh = plsc.VectorSubcoreMesh(core_axis_name="c", subcore_axis_name="s")

    @functools.partial(
        pl.kernel,
        out_type=jax.ShapeDtypeStruct((_TOTAL, _DIM), jnp.float32),
        mesh=mesh,
        scratch_types=[
            pltpu.VMEM((_CH,), jnp.int32),
            pltpu.VMEM((_CH,), jnp.int32),
            pltpu.VMEM((_CH, _QDIM), jnp.float32),
            pltpu.VMEM((_CH, _DIM), jnp.float32),
            pltpu.SemaphoreType.DMA,
        ],
    )
    def k(values_hbm, w0_hbm, w1_hbm, w2_hbm, w3_hbm, out_hbm, idx_v, q_v,
          rows_v, out_v, sem):
        w = lax.axis_index("s") * _NC + lax.axis_index("c")
        for t, w_hbm in enumerate((w0_hbm, w1_hbm, w2_hbm, w3_hbm)):
            for h in range(_B_PER_W // _CH):
                base = t * _PER_KEY + w * _B_PER_W + h * _CH
                pltpu.sync_copy(values_hbm.at[pl.ds(base, _CH)], idx_v)

                def qbody(b, _):
                    r = idx_v[pl.ds(b * 16, 16)]
                    q_v[pl.ds(b * 16, 16)] = ((r >> 15) << 13) + (r & 8191)
                    return 0

                lax.fori_loop(0, _CH // 16, qbody, 0)
                pltpu.async_copy(w_hbm.at[q_v], rows_v, sem).wait()

                def sbody(b, _):
                    offv = ((idx_v[pl.ds(b * 16, 16)] >> 13) & 3) * _DIM
                    for kk in range(16):
                        row = b * 16 + kk
                        off = offv[kk]
                        out_v[row, pl.ds(0, 16)] = rows_v[row, pl.ds(off, 16)]
                        out_v[row, pl.ds(16, 16)] = (
                            rows_v[row, pl.ds(off + 16, 16)]
                        )
                    return 0

                lax.fori_loop(0, _CH // 16, sbody, 0)
                pltpu.sync_copy(out_v, out_hbm.at[pl.ds(base, _CH)])

    return k(values, w0, w1, w2, w3)


@jax.jit
def kernel(values, W0, W1, W2, W3):
    return _grouped_gather(
        values,
        _relayout(W0.T),
        _relayout(W1.T),
        _relayout(W2.T),
        _relayout(W3.T),
    )


# speedup vs baseline: 1.6364x; 1.0037x over previous
import functools

import jax
import jax.numpy as jnp
from jax import lax
from jax.experimental import pallas as pl
from jax.experimental.pallas import tpu as pltpu
from jax.experimental.pallas import tpu_sc as plsc

_NUM_TABLES = 4
_VOCAB = 1000000
_DIM = 32
_PER_KEY = 16384
_TOTAL = _NUM_TABLES * _PER_KEY

_C = 32768
_C4 = _C // 4
_NB = (_VOCAB + _C - 1) // _C
_QROWS = _NB * _C4
_QDIM = 4 * _DIM

_info = plsc.get_sparse_core_info()
_NC, _NS = _info.num_cores, _info.num_subcores
_NW = _NC * _NS
_B_PER_W = _PER_KEY // _NW
_CH = _B_PER_W // 2


def _relayout(wt):
    def body(x_ref, o_ref):
        for m in range(4):
            o_ref[:, _DIM * m:_DIM * (m + 1)] = jnp.transpose(
                x_ref[:, m * _C4:(m + 1) * _C4]
            )

    return pl.pallas_call(
        body,
        grid=(_NB,),
        in_specs=[pl.BlockSpec((_DIM, _C), lambda g: (0, g))],
        out_specs=pl.BlockSpec((_C4, _QDIM), lambda g: (g, 0)),
        out_shape=jax.ShapeDtypeStruct((_QROWS, _QDIM), jnp.float32),
    )(wt)


def _grouped_gather(values, w0, w1, w2, w3):
    mesh = plsc.VectorSubcoreMesh(core_axis_name="c", subcore_axis_name="s")

    @functools.partial(
        pl.kernel,
        out_type=jax.ShapeDtypeStruct((_TOTAL, _DIM), jnp.float32),
        mesh=mesh,
        scratch_types=[
            pltpu.VMEM((_CH,), jnp.int32),
            pltpu.VMEM((_CH,), jnp.int32),
            pltpu.VMEM((_CH, _QDIM), jnp.float32),
            pltpu.VMEM((_CH, _DIM), jnp.float32),
            pltpu.SemaphoreType.DMA,
        ],
    )
    def k(values_hbm, w0_hbm, w1_hbm, w2_hbm, w3_hbm, out_hbm, idx_v, q_v,
          rows_v, out_v, sem):
        w = lax.axis_index("s") * _NC + lax.axis_index("c")
        for t, w_hbm in enumerate((w0_hbm, w1_hbm, w2_hbm, w3_hbm)):
            for h in range(_B_PER_W // _CH):
                base = t * _PER_KEY + w * _B_PER_W + h * _CH
                pltpu.sync_copy(values_hbm.at[pl.ds(base, _CH)], idx_v)

                def qbody(b, _):
                    r = idx_v[pl.ds(b * 16, 16)]
                    q_v[pl.ds(b * 16, 16)] = ((r >> 15) << 13) + (r & 8191)
                    return 0

                lax.fori_loop(0, _CH // 16, qbody, 0)
                pltpu.async_copy(w_hbm.at[q_v], rows_v, sem).wait()

                def sbody(b, _):
                    offv = ((idx_v[pl.ds(b * 16, 16)] >> 13) & 3) * _DIM
                    for kk in range(16):
                        row = b * 16 + kk
                        off = offv[kk]
                        out_v[row, pl.ds(0, 16)] = rows_v[row, pl.ds(off, 16)]
                        out_v[row, pl.ds(16, 16)] = (
                            rows_v[row, pl.ds(off + 16, 16)]
                        )
                    return 0

                lax.fori_loop(0, _CH // 16, sbody, 0)
                pltpu.sync_copy(out_v, out_hbm.at[pl.ds(base, _CH)])

    return k(values, w0, w1, w2, w3)


@jax.jit
def kernel(values, W0, W1, W2, W3):
    return _grouped_gather(
        values,
        _relayout(W0.T),
        _relayout(W1.T),
        _relayout(W2.T),
        _relayout(W3.T),
    )


# speedup vs baseline: 4.0608x; 2.4816x over previous
import functools

import jax
import jax.numpy as jnp
from jax import lax
from jax.experimental import pallas as pl
from jax.experimental.pallas import tpu as pltpu
from jax.experimental.pallas import tpu_sc as plsc

_NUM_TABLES = 4
_VOCAB = 1000000
_DIM = 32
_PER_KEY = 16384
_TOTAL = _NUM_TABLES * _PER_KEY

_C = 32768
_C4 = _C // 4
_NB = (_VOCAB + _C - 1) // _C
_QROWS = _NB * _C4
_QDIM = 4 * _DIM

_info = plsc.get_sparse_core_info()
_NC, _NS = _info.num_cores, _info.num_subcores
_NW = _NC * _NS
_B_PER_W = _PER_KEY // _NW
_CH = _B_PER_W // 2


def _relayout(wt):
    def body(x_ref, o_ref):
        z = jnp.concatenate(
            [x_ref[:, m * _C4:(m + 1) * _C4] for m in range(4)], axis=0
        )
        o_ref[...] = jnp.transpose(z)

    return pl.pallas_call(
        body,
        grid=(_NB,),
        in_specs=[pl.BlockSpec((_DIM, _C), lambda g: (0, g))],
        out_specs=pl.BlockSpec((_C4, _QDIM), lambda g: (g, 0)),
        out_shape=jax.ShapeDtypeStruct((_QROWS, _QDIM), jnp.float32),
    )(wt)


def _grouped_gather(values, w0, w1, w2, w3):
    mesh = plsc.VectorSubcoreMesh(core_axis_name="c", subcore_axis_name="s")

    @functools.partial(
        pl.kernel,
        out_type=jax.ShapeDtypeStruct((_TOTAL, _DIM), jnp.float32),
        mesh=mesh,
        scratch_types=[
            pltpu.VMEM((_CH,), jnp.int32),
            pltpu.VMEM((_CH,), jnp.int32),
            pltpu.VMEM((_CH, _QDIM), jnp.float32),
            pltpu.VMEM((_CH, _DIM), jnp.float32),
            pltpu.SemaphoreType.DMA,
        ],
    )
    def k(values_hbm, w0_hbm, w1_hbm, w2_hbm, w3_hbm, out_hbm, idx_v, q_v,
          rows_v, out_v, sem):
        w = lax.axis_index("s") * _NC + lax.axis_index("c")
        for t, w_hbm in enumerate((w0_hbm, w1_hbm, w2_hbm, w3_hbm)):
            for h in range(_B_PER_W // _CH):
                base = t * _PER_KEY + w * _B_PER_W + h * _CH
                pltpu.sync_copy(values_hbm.at[pl.ds(base, _CH)], idx_v)

                def qbody(b, _):
                    r = idx_v[pl.ds(b * 16, 16)]
                    q_v[pl.ds(b * 16, 16)] = ((r >> 15) << 13) + (r & 8191)
                    return 0

                lax.fori_loop(0, _CH // 16, qbody, 0)
                pltpu.async_copy(w_hbm.at[q_v], rows_v, sem).wait()

                def sbody(b, _):
                    offv = ((idx_v[pl.ds(b * 16, 16)] >> 13) & 3) * _DIM
                    for kk in range(16):
                        row = b * 16 + kk
                        off = offv[kk]
                        out_v[row, pl.ds(0, 16)] = rows_v[row, pl.ds(off, 16)]
                        out_v[row, pl.ds(16, 16)] = (
                            rows_v[row, pl.ds(off + 16, 16)]
                        )
                    return 0

                lax.fori_loop(0, _CH // 16, sbody, 0)
                pltpu.sync_copy(out_v, out_hbm.at[pl.ds(base, _CH)])

    return k(values, w0, w1, w2, w3)


@jax.jit
def kernel(values, W0, W1, W2, W3):
    return _grouped_gather(
        values,
        _relayout(W0.T),
        _relayout(W1.T),
        _relayout(W2.T),
        _relayout(W3.T),
    )


# speedup vs baseline: 4.0933x; 1.0080x over previous
import functools

import jax
import jax.numpy as jnp
from jax import lax
from jax.experimental import pallas as pl
from jax.experimental.pallas import tpu as pltpu
from jax.experimental.pallas import tpu_sc as plsc

_NUM_TABLES = 4
_VOCAB = 1000000
_DIM = 32
_PER_KEY = 16384
_TOTAL = _NUM_TABLES * _PER_KEY

_C = 65536
_C4 = _C // 4
_NB = (_VOCAB + _C - 1) // _C
_QROWS = _NB * _C4
_QDIM = 4 * _DIM

_info = plsc.get_sparse_core_info()
_NC, _NS = _info.num_cores, _info.num_subcores
_NW = _NC * _NS
_B_PER_W = _PER_KEY // _NW
_CH = _B_PER_W // 2


def _relayout(wt):
    def body(x_ref, o_ref):
        z = jnp.concatenate(
            [x_ref[:, m * _C4:(m + 1) * _C4] for m in range(4)], axis=0
        )
        o_ref[...] = jnp.transpose(z)

    return pl.pallas_call(
        body,
        grid=(_NB,),
        in_specs=[pl.BlockSpec((_DIM, _C), lambda g: (0, g))],
        out_specs=pl.BlockSpec((_C4, _QDIM), lambda g: (g, 0)),
        out_shape=jax.ShapeDtypeStruct((_QROWS, _QDIM), jnp.float32),
    )(wt)


def _grouped_gather(values, w0, w1, w2, w3):
    mesh = plsc.VectorSubcoreMesh(core_axis_name="c", subcore_axis_name="s")

    @functools.partial(
        pl.kernel,
        out_type=jax.ShapeDtypeStruct((_TOTAL, _DIM), jnp.float32),
        mesh=mesh,
        scratch_types=[
            pltpu.VMEM((_CH,), jnp.int32),
            pltpu.VMEM((_CH,), jnp.int32),
            pltpu.VMEM((_CH, _QDIM), jnp.float32),
            pltpu.VMEM((_CH, _DIM), jnp.float32),
            pltpu.SemaphoreType.DMA,
        ],
    )
    def k(values_hbm, w0_hbm, w1_hbm, w2_hbm, w3_hbm, out_hbm, idx_v, q_v,
          rows_v, out_v, sem):
        w = lax.axis_index("s") * _NC + lax.axis_index("c")
        for t, w_hbm in enumerate((w0_hbm, w1_hbm, w2_hbm, w3_hbm)):
            for h in range(_B_PER_W // _CH):
                base = t * _PER_KEY + w * _B_PER_W + h * _CH
                pltpu.sync_copy(values_hbm.at[pl.ds(base, _CH)], idx_v)

                def qbody(b, _):
                    r = idx_v[pl.ds(b * 16, 16)]
                    q_v[pl.ds(b * 16, 16)] = ((r >> 16) << 14) + (r & 16383)
                    return 0

                lax.fori_loop(0, _CH // 16, qbody, 0)
                pltpu.async_copy(w_hbm.at[q_v], rows_v, sem).wait()

                def sbody(b, _):
                    offv = ((idx_v[pl.ds(b * 16, 16)] >> 14) & 3) * _DIM
                    for kk in range(16):
                        row = b * 16 + kk
                        off = offv[kk]
                        out_v[row, pl.ds(0, 16)] = rows_v[row, pl.ds(off, 16)]
                        out_v[row, pl.ds(16, 16)] = (
                            rows_v[row, pl.ds(off + 16, 16)]
                        )
                    return 0

                lax.fori_loop(0, _CH // 16, sbody, 0)
                pltpu.sync_copy(out_v, out_hbm.at[pl.ds(base, _CH)])

    return k(values, w0, w1, w2, w3)


@jax.jit
def kernel(values, W0, W1, W2, W3):
    return _grouped_gather(
        values,
        _relayout(W0.T),
        _relayout(W1.T),
        _relayout(W2.T),
        _relayout(W3.T),
    )


# speedup vs baseline: 4.3507x; 1.0629x over previous
import functools

import jax
import jax.numpy as jnp
from jax import lax
from jax.experimental import pallas as pl
from jax.experimental.pallas import tpu as pltpu
from jax.experimental.pallas import tpu_sc as plsc

_NUM_TABLES = 4
_VOCAB = 1000000
_DIM = 32
_PER_KEY = 16384
_TOTAL = _NUM_TABLES * _PER_KEY

_C = 65536
_C4 = _C // 4
_NB = (_VOCAB + _C - 1) // _C
_QROWS = _NB * _C4
_QDIM = 4 * _DIM

_info = plsc.get_sparse_core_info()
_NC, _NS = _info.num_cores, _info.num_subcores
_NW = _NC * _NS
_B_PER_W = _PER_KEY // _NW
_CH = _B_PER_W // 2


def _relayout(wt):
    def body(x_ref, o_ref):
        z = jnp.concatenate(
            [x_ref[:, m * _C4:(m + 1) * _C4] for m in range(4)], axis=0
        )
        o_ref[...] = jnp.transpose(z)

    return pl.pallas_call(
        body,
        grid=(_NB,),
        in_specs=[pl.BlockSpec((_DIM, _C), lambda g: (0, g))],
        out_specs=pl.BlockSpec((_C4, _QDIM), lambda g: (g, 0)),
        out_shape=jax.ShapeDtypeStruct((_QROWS, _QDIM), jnp.float32),
    )(wt)


def _gather_one(values, scratch):
    mesh = plsc.VectorSubcoreMesh(core_axis_name="c", subcore_axis_name="s")

    @functools.partial(
        pl.kernel,
        out_type=jax.ShapeDtypeStruct((_PER_KEY, _DIM), jnp.float32),
        mesh=mesh,
        scratch_types=[
            pltpu.VMEM((_CH,), jnp.int32),
            pltpu.VMEM((_CH,), jnp.int32),
            pltpu.VMEM((_CH, _QDIM), jnp.float32),
            pltpu.VMEM((_CH, _DIM), jnp.float32),
            pltpu.SemaphoreType.DMA,
        ],
    )
    def k(values_hbm, w_hbm, out_hbm, idx_v, q_v, rows_v, out_v, sem):
        w = lax.axis_index("s") * _NC + lax.axis_index("c")
        for h in range(_B_PER_W // _CH):
            base = w * _B_PER_W + h * _CH
            pltpu.sync_copy(values_hbm.at[pl.ds(base, _CH)], idx_v)

            def qbody(b, _):
                r = idx_v[pl.ds(b * 16, 16)]
                q_v[pl.ds(b * 16, 16)] = ((r >> 16) << 14) + (r & 16383)
                return 0

            lax.fori_loop(0, _CH // 16, qbody, 0)
            pltpu.async_copy(w_hbm.at[q_v], rows_v, sem).wait()

            def sbody(b, _):
                offv = ((idx_v[pl.ds(b * 16, 16)] >> 14) & 3) * _DIM
                for kk in range(16):
                    row = b * 16 + kk
                    off = offv[kk]
                    out_v[row, pl.ds(0, 16)] = rows_v[row, pl.ds(off, 16)]
                    out_v[row, pl.ds(16, 16)] = rows_v[row, pl.ds(off + 16, 16)]
                return 0

            lax.fori_loop(0, _CH // 16, sbody, 0)
            pltpu.sync_copy(out_v, out_hbm.at[pl.ds(base, _CH)])

    return k(values, scratch)


@jax.jit
def kernel(values, W0, W1, W2, W3):
    outs = []
    for t, w in enumerate((W0, W1, W2, W3)):
        vals_t = values[t * _PER_KEY:(t + 1) * _PER_KEY]
        outs.append(_gather_one(vals_t, _relayout(w.T)))
    return jnp.concatenate(outs, axis=0)
